# Initial kernel scaffold; baseline (speedup 1.0000x reference)
import functools

import jax
import jax.numpy as jnp
from jax import lax
from jax.experimental import pallas as pl
from jax.experimental.pallas import tpu as pltpu
from jax.experimental.pallas import tpu_sc as plsc

N = 10000
R = 10240
DF = 128
H = 256
HH = 128
C = 10
CP = 16
E_OLD = 320000
E_NEW = 156000
NT = 16
NCH_OLD = 157
NCH_NEW = 77
CH = 128
STRIPE = R // NT
BR = 512
GR = R // BR

_mesh = plsc.VectorSubcoreMesh(core_axis_name="c", subcore_axis_name="s")
_f32 = jnp.float32
_i32 = jnp.int32


def _zero_gbuf(gbuf, nrow, ncol16):
    def body(i, _):
        for k in range(ncol16):
            gbuf[i, pl.ds(k * 16, 16)] = jnp.zeros((16,), _f32)
        return 0
    lax.fori_loop(0, nrow, body, 0)


@functools.partial(
    pl.kernel,
    out_type=[
        jax.ShapeDtypeStruct((2, R), _f32),
        jax.ShapeDtypeStruct((NT, NCH_NEW, CH), _f32),
    ],
    mesh=_mesh,
    scratch_types=[
        pltpu.VMEM((NCH_OLD, CH), _i32),
        pltpu.VMEM((NCH_NEW, CH), _i32),
        pltpu.VMEM((NCH_NEW, CH), _f32),
        pltpu.VMEM((CH,), _f32),
        pltpu.VMEM((CH,), _f32),
        pltpu.VMEM((STRIPE,), _f32),
        pltpu.VMEM_SHARED((R,), _f32),
    ],
)
def _deg_w_kernel(odst_h, ndst_h, probs_h, degp_h, w_h,
                  odst, ndst, pslab, ones_v, wbuf, zbuf, deg_sp):
    c = lax.axis_index("c")
    s = lax.axis_index("s")

    def z16(i, _):
        zbuf[pl.ds(i * 16, 16)] = jnp.zeros((16,), _f32)
        return 0
    lax.fori_loop(0, STRIPE // 16, z16, 0)
    pltpu.sync_copy(zbuf, deg_sp.at[pl.ds(s * STRIPE, STRIPE)])
    for k in range(CH // 16):
        ones_v[pl.ds(k * 16, 16)] = jnp.ones((16,), _f32)
    pltpu.sync_copy(odst_h.at[s], odst)
    pltpu.sync_copy(ndst_h.at[s], ndst)
    pltpu.sync_copy(probs_h.at[s], pslab)
    plsc.subcore_barrier()

    ostart = c * 79
    ocnt = 79 - c

    def old_body(j, _):
        pltpu.sync_copy(ones_v, deg_sp.at[odst.at[j]], add=True)
        return 0
    lax.fori_loop(ostart, ostart + ocnt, old_body, 0)

    nstart = c * 39
    ncnt = 39 - c

    def new_body(j, _):
        for k in range(CH // 16):
            p = pslab[j, pl.ds(k * 16, 16)]
            wbuf[pl.ds(k * 16, 16)] = 1.0 / (1.0 + jnp.exp(-p))
        pltpu.sync_copy(wbuf, w_h.at[s, j])
        pltpu.sync_copy(wbuf, deg_sp.at[ndst.at[j]], add=True)
        return 0
    lax.fori_loop(nstart, nstart + ncnt, new_body, 0)

    plsc.subcore_barrier()
    pltpu.sync_copy(deg_sp.at[pl.ds(s * STRIPE, STRIPE)],
                    degp_h.at[c, pl.ds(s * STRIPE, STRIPE)])


@functools.partial(
    pl.kernel,
    out_type=jax.ShapeDtypeStruct((2 * R, HH), _f32),
    mesh=_mesh,
    scratch_types=[
        pltpu.VMEM((NCH_OLD, CH), _i32),
        pltpu.VMEM((NCH_OLD, CH), _i32),
        pltpu.VMEM((NCH_NEW, CH), _i32),
        pltpu.VMEM((NCH_NEW, CH), _i32),
        pltpu.VMEM((NCH_NEW, CH), _f32),
        pltpu.VMEM((CH, HH), _f32),
        pltpu.VMEM_SHARED((R, HH), _f32),
        pltpu.SemaphoreType.DMA,
    ],
)
def _layer_half_kernel(z_h, osrc_h, odst_h, nsrc_h, ndst_h, w_h, acc_h,
                       osrc, odst, nsrc, ndst, wsl, gbuf, out_sp, sem):
    c = lax.axis_index("c")
    s = lax.axis_index("s")
    off = c * R

    _zero_gbuf(gbuf, CH, HH // 16)
    for t in range(STRIPE // CH):
        pltpu.sync_copy(gbuf, out_sp.at[pl.ds(s * STRIPE + t * CH, CH)])

    pltpu.sync_copy(osrc_h.at[s], osrc)
    pltpu.sync_copy(odst_h.at[s], odst)
    pltpu.sync_copy(nsrc_h.at[s], nsrc)
    pltpu.sync_copy(ndst_h.at[s], ndst)
    pltpu.sync_copy(w_h.at[s], wsl)

    def off_old(j, _):
        for k in range(CH // 16):
            osrc[j, pl.ds(k * 16, 16)] = osrc[j, pl.ds(k * 16, 16)] + off
        return 0
    lax.fori_loop(0, NCH_OLD, off_old, 0)

    def off_new(j, _):
        for k in range(CH // 16):
            nsrc[j, pl.ds(k * 16, 16)] = nsrc[j, pl.ds(k * 16, 16)] + off
        return 0
    lax.fori_loop(0, NCH_NEW, off_new, 0)
    plsc.subcore_barrier()

    def old_body(j, _):
        pltpu.async_copy(z_h.at[osrc.at[j]], gbuf, sem).wait()
        pltpu.sync_copy(gbuf, out_sp.at[odst.at[j]], add=True)
        return 0
    lax.fori_loop(0, NCH_OLD, old_body, 0)

    def new_body(j, _):
        pltpu.async_copy(z_h.at[nsrc.at[j]], gbuf, sem).wait()

        def rowm(i, _):
            sw = wsl[j, i]
            for k in range(HH // 16):
                gbuf[i, pl.ds(k * 16, 16)] = gbuf[i, pl.ds(k * 16, 16)] * sw
            return 0
        lax.fori_loop(0, CH, rowm, 0)
        pltpu.sync_copy(gbuf, out_sp.at[ndst.at[j]], add=True)
        return 0
    lax.fori_loop(0, NCH_NEW, new_body, 0)

    plsc.subcore_barrier()
    pltpu.sync_copy(out_sp.at[pl.ds(s * STRIPE, STRIPE)],
                    acc_h.at[pl.ds(off + s * STRIPE, STRIPE)])


@functools.partial(
    pl.kernel,
    out_type=jax.ShapeDtypeStruct((2, R, CP), _f32),
    mesh=_mesh,
    scratch_types=[
        pltpu.VMEM((NCH_OLD, CH), _i32),
        pltpu.VMEM((NCH_OLD, CH), _i32),
        pltpu.VMEM((NCH_NEW, CH), _i32),
        pltpu.VMEM((NCH_NEW, CH), _i32),
        pltpu.VMEM((NCH_NEW, CH), _f32),
        pltpu.VMEM((CH, CP), _f32),
        pltpu.VMEM_SHARED((R, CP), _f32),
        pltpu.SemaphoreType.DMA,
    ],
)
def _l3_kernel(z_h, osrc_h, odst_h, nsrc_h, ndst_h, w_h, acc_h,
               osrc, odst, nsrc, ndst, wsl, gbuf, out_sp, sem):
    c = lax.axis_index("c")
    s = lax.axis_index("s")

    _zero_gbuf(gbuf, CH, CP // 16)
    for t in range(STRIPE // CH):
        pltpu.sync_copy(gbuf, out_sp.at[pl.ds(s * STRIPE + t * CH, CH)])

    pltpu.sync_copy(osrc_h.at[s], osrc)
    pltpu.sync_copy(odst_h.at[s], odst)
    pltpu.sync_copy(nsrc_h.at[s], nsrc)
    pltpu.sync_copy(ndst_h.at[s], ndst)
    pltpu.sync_copy(w_h.at[s], wsl)
    plsc.subcore_barrier()

    ostart = c * 79
    ocnt = 79 - c

    def old_body(j, _):
        pltpu.async_copy(z_h.at[osrc.at[j]], gbuf, sem).wait()
        pltpu.sync_copy(gbuf, out_sp.at[odst.at[j]], add=True)
        return 0
    lax.fori_loop(ostart, ostart + ocnt, old_body, 0)

    nstart = c * 39
    ncnt = 39 - c

    def new_body(j, _):
        pltpu.async_copy(z_h.at[nsrc.at[j]], gbuf, sem).wait()

        def rowm(i, _):
            sw = wsl[j, i]
            gbuf[i, pl.ds(0, 16)] = gbuf[i, pl.ds(0, 16)] * sw
            return 0
        lax.fori_loop(0, CH, rowm, 0)
        pltpu.sync_copy(gbuf, out_sp.at[ndst.at[j]], add=True)
        return 0
    lax.fori_loop(nstart, nstart + ncnt, new_body, 0)

    plsc.subcore_barrier()
    pltpu.sync_copy(out_sp.at[pl.ds(s * STRIPE, STRIPE)],
                    acc_h.at[c, pl.ds(s * STRIPE, STRIPE)])


def _tc_dinv_body(degp_ref, dinv_ref):
    d = degp_ref[0] + degp_ref[1] + 1.0
    row = lax.broadcasted_iota(_i32, (R // 128, 128), 0)
    col = lax.broadcasted_iota(_i32, (R // 128, 128), 1)
    valid = (row * 128 + col) < N
    dinv_ref[...] = jnp.where(valid, lax.rsqrt(d), 0.0)


def _tc_dinv(degp):
    return pl.pallas_call(
        _tc_dinv_body,
        out_shape=jax.ShapeDtypeStruct((R // 128, 128), _f32),
    )(degp.reshape(2, R // 128, 128))


def _tc_l1_body(x_ref, w_ref, dinv_ref, o_ref):
    o_ref[...] = jnp.dot(x_ref[...], w_ref[...],
                         preferred_element_type=_f32) * dinv_ref[...]


def _tc_l1(x_pad, W1, dinv_col):
    return pl.pallas_call(
        _tc_l1_body,
        grid=(GR, 2),
        in_specs=[
            pl.BlockSpec((BR, DF), lambda r, c: (r, 0)),
            pl.BlockSpec((DF, HH), lambda r, c: (0, c)),
            pl.BlockSpec((BR, 1), lambda r, c: (r, 0)),
        ],
        out_specs=pl.BlockSpec((BR, HH), lambda r, c: (c * GR + r, 0)),
        out_shape=jax.ShapeDtypeStruct((2 * R, HH), _f32),
    )(x_pad, W1, dinv_col)


def _tc_fused_body(a_ref, z_ref, dinv_ref, b_ref, w_ref, o_ref):
    acc = jnp.concatenate([a_ref[0], a_ref[1]], axis=1)
    zc = jnp.concatenate([z_ref[0], z_ref[1]], axis=1)
    dinv = dinv_ref[...]
    act = jnp.maximum(dinv * (acc + zc) + b_ref[...], 0.0)
    o_ref[...] = jnp.dot(act, w_ref[...], preferred_element_type=_f32) * dinv


def _tc_fused(acc, zp, dinv_col, b, W2):
    return pl.pallas_call(
        _tc_fused_body,
        grid=(GR, 2),
        in_specs=[
            pl.BlockSpec((2, BR, HH), lambda r, c: (0, r, 0)),
            pl.BlockSpec((2, BR, HH), lambda r, c: (0, r, 0)),
            pl.BlockSpec((BR, 1), lambda r, c: (r, 0)),
            pl.BlockSpec((1, H), lambda r, c: (0, 0)),
            pl.BlockSpec((H, HH), lambda r, c: (0, c)),
        ],
        out_specs=pl.BlockSpec((BR, HH), lambda r, c: (c * GR + r, 0)),
        out_shape=jax.ShapeDtypeStruct((2 * R, HH), _f32),
    )(acc.reshape(2, R, HH), zp.reshape(2, R, HH), dinv_col,
      b.reshape(1, H), W2)


def _tc_fused3(acc, zp, dinv_col, b, W3p):
    return pl.pallas_call(
        _tc_fused_body,
        grid=(GR, 1),
        in_specs=[
            pl.BlockSpec((2, BR, HH), lambda r, c: (0, r, 0)),
            pl.BlockSpec((2, BR, HH), lambda r, c: (0, r, 0)),
            pl.BlockSpec((BR, 1), lambda r, c: (r, 0)),
            pl.BlockSpec((1, H), lambda r, c: (0, 0)),
            pl.BlockSpec((H, CP), lambda r, c: (0, 0)),
        ],
        out_specs=pl.BlockSpec((BR, CP), lambda r, c: (r, 0)),
        out_shape=jax.ShapeDtypeStruct((R, CP), _f32),
    )(acc.reshape(2, R, HH), zp.reshape(2, R, HH), dinv_col,
      b.reshape(1, H), W3p)


def _tc_final_body(a_ref, z_ref, dinv_ref, b_ref, o_ref):
    sgt = dinv_ref[...] * (a_ref[0] + a_ref[1] + z_ref[...]) + b_ref[...]
    col = lax.broadcasted_iota(_i32, (BR, CP), 1)
    sm = jnp.where(col < C, sgt, -1e30)
    m = jnp.max(sm, axis=1, keepdims=True)
    e = jnp.where(col < C, jnp.exp(sm - m), 0.0)
    lse = jnp.log(jnp.sum(e, axis=1, keepdims=True))
    o_ref[...] = sm - m - lse


def _tc_final(acc3, z3p, dinv_col, b3p):
    return pl.pallas_call(
        _tc_final_body,
        grid=(GR,),
        in_specs=[
            pl.BlockSpec((2, BR, CP), lambda r: (0, r, 0)),
            pl.BlockSpec((BR, CP), lambda r: (r, 0)),
            pl.BlockSpec((BR, 1), lambda r: (r, 0)),
            pl.BlockSpec((1, CP), lambda r: (0, 0)),
        ],
        out_specs=pl.BlockSpec((BR, CP), lambda r: (r, 0)),
        out_shape=jax.ShapeDtypeStruct((R, CP), _f32),
    )(acc3, z3p, dinv_col, b3p)


def _slab(v, nch, pad_val):
    per = v.shape[0] // NT
    v = v.reshape(NT, per)
    v = jnp.pad(v, ((0, 0), (0, nch * CH - per)), constant_values=pad_val)
    return v.reshape(NT, nch, CH)


def kernel(x, old_edge_index, new_edges, edge_probs, W1, b1, W2, b2, W3, b3):
    osrc = _slab(old_edge_index[0], NCH_OLD, N)
    odst = _slab(old_edge_index[1], NCH_OLD, N)
    nsrc = _slab(new_edges[0], NCH_NEW, N)
    ndst = _slab(new_edges[1], NCH_NEW, N)
    pslab = _slab(edge_probs, NCH_NEW, 0.0)

    x_pad = jnp.pad(x, ((0, R - N), (0, 0)))
    W3p = jnp.pad(W3, ((0, 0), (0, CP - C)))
    b3p = jnp.pad(b3, (0, CP - C)).reshape(1, CP)

    degp, wslab = _deg_w_kernel(odst, ndst, pslab)
    dinv_col = _tc_dinv(degp).reshape(R, 1)

    z1 = _tc_l1(x_pad, W1, dinv_col)
    a1 = _layer_half_kernel(z1, osrc, odst, nsrc, ndst, wslab)
    z2 = _tc_fused(a1, z1, dinv_col, b1, W2)
    a2 = _layer_half_kernel(z2, osrc, odst, nsrc, ndst, wslab)
    z3 = _tc_fused3(a2, z2, dinv_col, b2, W3p)
    a3 = _l3_kernel(z3, osrc, odst, nsrc, ndst, wslab)
    lp = _tc_final(a3, z3, dinv_col, b3p)

    log_probs = lp[:N, :C]
    updated_edge_index = jnp.concatenate([old_edge_index, new_edges], axis=1)
    return (log_probs, updated_edge_index)


# speedup vs baseline: 5.5289x; 5.5289x over previous
import functools

import jax
import jax.numpy as jnp
from jax import lax
from jax.experimental import pallas as pl
from jax.experimental.pallas import tpu as pltpu
from jax.experimental.pallas import tpu_sc as plsc

N = 10000
R = 10240
DF = 128
H = 256
HH = 128
C = 10
CP = 16
E_OLD = 320000
E_NEW = 156000
NT = 16
NCH_OLD = 160
NCH_NEW = 80
WG = 8
CH = 128
STRIPE = R // NT
BR = 512
GR = R // BR

_mesh = plsc.VectorSubcoreMesh(core_axis_name="c", subcore_axis_name="s")
_f32 = jnp.float32
_i32 = jnp.int32


def _zero_gbuf(gbuf, nrow, ncol16):
    def body(i, _):
        for k in range(ncol16):
            gbuf[i, pl.ds(k * 16, 16)] = jnp.zeros((16,), _f32)
        return 0
    lax.fori_loop(0, nrow, body, 0)


@functools.partial(
    pl.kernel,
    out_type=[
        jax.ShapeDtypeStruct((2, R), _f32),
        jax.ShapeDtypeStruct((NT, NCH_NEW, CH), _f32),
    ],
    mesh=_mesh,
    scratch_types=[
        pltpu.VMEM((NCH_OLD, CH), _i32),
        pltpu.VMEM((NCH_NEW, CH), _i32),
        pltpu.VMEM((NCH_NEW, CH), _f32),
        pltpu.VMEM((CH,), _f32),
        pltpu.VMEM((CH,), _f32),
        pltpu.VMEM((STRIPE,), _f32),
        pltpu.VMEM_SHARED((R,), _f32),
    ],
)
def _deg_w_kernel(odst_h, ndst_h, probs_h, degp_h, w_h,
                  odst, ndst, pslab, ones_v, wbuf, zbuf, deg_sp):
    c = lax.axis_index("c")
    s = lax.axis_index("s")

    def z16(i, _):
        zbuf[pl.ds(i * 16, 16)] = jnp.zeros((16,), _f32)
        return 0
    lax.fori_loop(0, STRIPE // 16, z16, 0)
    pltpu.sync_copy(zbuf, deg_sp.at[pl.ds(s * STRIPE, STRIPE)])
    for k in range(CH // 16):
        ones_v[pl.ds(k * 16, 16)] = jnp.ones((16,), _f32)
    pltpu.sync_copy(odst_h.at[s], odst)
    pltpu.sync_copy(ndst_h.at[s], ndst)
    pltpu.sync_copy(probs_h.at[s], pslab)
    plsc.subcore_barrier()

    ostart = c * (NCH_OLD // 2)

    def old_body(j, _):
        pltpu.sync_copy(ones_v, deg_sp.at[odst.at[j]], add=True)
        return 0
    lax.fori_loop(ostart, ostart + NCH_OLD // 2, old_body, 0)

    nstart = c * (NCH_NEW // 2)

    def new_body(j, _):
        for k in range(CH // 16):
            p = pslab[j, pl.ds(k * 16, 16)]
            wbuf[pl.ds(k * 16, 16)] = 1.0 / (1.0 + jnp.exp(-p))
        pltpu.sync_copy(wbuf, w_h.at[s, j])
        pltpu.sync_copy(wbuf, deg_sp.at[ndst.at[j]], add=True)
        return 0
    lax.fori_loop(nstart, nstart + NCH_NEW // 2, new_body, 0)

    plsc.subcore_barrier()
    pltpu.sync_copy(deg_sp.at[pl.ds(s * STRIPE, STRIPE)],
                    degp_h.at[c, pl.ds(s * STRIPE, STRIPE)])


@functools.partial(
    pl.kernel,
    out_type=jax.ShapeDtypeStruct((2 * R, HH), _f32),
    mesh=_mesh,
    scratch_types=[
        pltpu.VMEM((WG, CH), _i32),
        pltpu.VMEM((WG, CH), _i32),
        pltpu.VMEM((WG, CH), _f32),
        pltpu.VMEM((CH, HH), _f32),
        pltpu.VMEM_SHARED((R, HH), _f32),
        pltpu.SemaphoreType.DMA,
    ],
)
def _layer_half_kernel(z_h, osrc_h, odst_h, nsrc_h, ndst_h, w_h, acc_h,
                       srcw, dstw, ww, gbuf, out_sp, sem):
    c = lax.axis_index("c")
    s = lax.axis_index("s")
    off = c * R

    _zero_gbuf(gbuf, CH, HH // 16)
    for t in range(STRIPE // CH):
        pltpu.sync_copy(gbuf, out_sp.at[pl.ds(s * STRIPE + t * CH, CH)])
    plsc.subcore_barrier()

    def add_off(_, __):
        def body(j, ___):
            for k in range(CH // 16):
                srcw[j, pl.ds(k * 16, 16)] = srcw[j, pl.ds(k * 16, 16)] + off
            return 0
        lax.fori_loop(0, WG, body, 0)

    def old_win(wi, _):
        pltpu.sync_copy(osrc_h.at[s, pl.ds(wi * WG, WG)], srcw)
        pltpu.sync_copy(odst_h.at[s, pl.ds(wi * WG, WG)], dstw)
        add_off(None, None)

        def chunk(j, __):
            pltpu.async_copy(z_h.at[srcw.at[j]], gbuf, sem).wait()
            pltpu.sync_copy(gbuf, out_sp.at[dstw.at[j]], add=True)
            return 0
        lax.fori_loop(0, WG, chunk, 0)
        return 0
    lax.fori_loop(0, NCH_OLD // WG, old_win, 0)

    def new_win(wi, _):
        pltpu.sync_copy(nsrc_h.at[s, pl.ds(wi * WG, WG)], srcw)
        pltpu.sync_copy(ndst_h.at[s, pl.ds(wi * WG, WG)], dstw)
        pltpu.sync_copy(w_h.at[s, pl.ds(wi * WG, WG)], ww)
        add_off(None, None)

        def chunk(j, __):
            pltpu.async_copy(z_h.at[srcw.at[j]], gbuf, sem).wait()

            def rowm(g, ___):
                wv = ww[j, pl.ds(g * 16, 16)]
                for t in range(16):
                    i = g * 16 + t
                    sw = wv[t]
                    for k in range(HH // 16):
                        gbuf[i, pl.ds(k * 16, 16)] = (
                            gbuf[i, pl.ds(k * 16, 16)] * sw)
                return 0
            lax.fori_loop(0, CH // 16, rowm, 0)
            pltpu.sync_copy(gbuf, out_sp.at[dstw.at[j]], add=True)
            return 0
        lax.fori_loop(0, WG, chunk, 0)
        return 0
    lax.fori_loop(0, NCH_NEW // WG, new_win, 0)

    plsc.subcore_barrier()
    pltpu.sync_copy(out_sp.at[pl.ds(s * STRIPE, STRIPE)],
                    acc_h.at[pl.ds(off + s * STRIPE, STRIPE)])


@functools.partial(
    pl.kernel,
    out_type=jax.ShapeDtypeStruct((2 * R, HH), _f32),
    mesh=_mesh,
    scratch_types=[
        pltpu.VMEM((WG, CH), _i32),
        pltpu.VMEM((WG, CH), _i32),
        pltpu.VMEM((WG, CH), _f32),
        pltpu.VMEM((CH, HH), _f32),
        pltpu.VMEM_SHARED((R, HH), _f32),
        pltpu.SemaphoreType.DMA,
    ],
)
def _l3_kernel(z_h, osrc_h, odst_h, nsrc_h, ndst_h, w_h, acc_h,
               srcw, dstw, ww, gbuf, out_sp, sem):
    c = lax.axis_index("c")
    s = lax.axis_index("s")
    off = c * R

    _zero_gbuf(gbuf, CH, HH // 16)
    for t in range(STRIPE // CH):
        pltpu.sync_copy(gbuf, out_sp.at[pl.ds(s * STRIPE + t * CH, CH)])
    plsc.subcore_barrier()

    ow0 = c * (NCH_OLD // WG // 2)

    def old_win(wi, _):
        pltpu.sync_copy(osrc_h.at[s, pl.ds(wi * WG, WG)], srcw)
        pltpu.sync_copy(odst_h.at[s, pl.ds(wi * WG, WG)], dstw)

        def chunk(j, __):
            pltpu.async_copy(z_h.at[srcw.at[j]], gbuf, sem).wait()
            pltpu.sync_copy(gbuf, out_sp.at[dstw.at[j]], add=True)
            return 0
        lax.fori_loop(0, WG, chunk, 0)
        return 0
    lax.fori_loop(ow0, ow0 + NCH_OLD // WG // 2, old_win, 0)

    nw0 = c * (NCH_NEW // WG // 2)

    def new_win(wi, _):
        pltpu.sync_copy(nsrc_h.at[s, pl.ds(wi * WG, WG)], srcw)
        pltpu.sync_copy(ndst_h.at[s, pl.ds(wi * WG, WG)], dstw)
        pltpu.sync_copy(w_h.at[s, pl.ds(wi * WG, WG)], ww)

        def chunk(j, __):
            pltpu.async_copy(z_h.at[srcw.at[j]], gbuf, sem).wait()

            def rowm(g, ___):
                wv = ww[j, pl.ds(g * 16, 16)]
                for t in range(16):
                    i = g * 16 + t
                    gbuf[i, pl.ds(0, 16)] = gbuf[i, pl.ds(0, 16)] * wv[t]
                return 0
            lax.fori_loop(0, CH // 16, rowm, 0)
            pltpu.sync_copy(gbuf, out_sp.at[dstw.at[j]], add=True)
            return 0
        lax.fori_loop(0, WG, chunk, 0)
        return 0
    lax.fori_loop(nw0, nw0 + NCH_NEW // WG // 2, new_win, 0)

    plsc.subcore_barrier()
    pltpu.sync_copy(out_sp.at[pl.ds(s * STRIPE, STRIPE)],
                    acc_h.at[pl.ds(off + s * STRIPE, STRIPE)])


def _tc_dinv_body(degp_ref, dinv_ref):
    d = degp_ref[0] + degp_ref[1] + 1.0
    row = lax.broadcasted_iota(_i32, (R // 128, 128), 0)
    col = lax.broadcasted_iota(_i32, (R // 128, 128), 1)
    valid = (row * 128 + col) < N
    dinv_ref[...] = jnp.where(valid, lax.rsqrt(d), 0.0)


def _tc_dinv(degp):
    return pl.pallas_call(
        _tc_dinv_body,
        out_shape=jax.ShapeDtypeStruct((R // 128, 128), _f32),
    )(degp.reshape(2, R // 128, 128))


def _tc_l1_body(x_ref, w_ref, dinv_ref, o_ref):
    o_ref[...] = jnp.dot(x_ref[...], w_ref[...],
                         preferred_element_type=_f32) * dinv_ref[...]


def _tc_l1(x_pad, W1, dinv_col):
    return pl.pallas_call(
        _tc_l1_body,
        grid=(GR, 2),
        in_specs=[
            pl.BlockSpec((BR, DF), lambda r, c: (r, 0)),
            pl.BlockSpec((DF, HH), lambda r, c: (0, c)),
            pl.BlockSpec((BR, 1), lambda r, c: (r, 0)),
        ],
        out_specs=pl.BlockSpec((BR, HH), lambda r, c: (c * GR + r, 0)),
        out_shape=jax.ShapeDtypeStruct((2 * R, HH), _f32),
    )(x_pad, W1, dinv_col)


def _tc_fused_body(a_ref, z_ref, dinv_ref, b_ref, w_ref, o_ref):
    acc = jnp.concatenate([a_ref[0], a_ref[1]], axis=1)
    zc = jnp.concatenate([z_ref[0], z_ref[1]], axis=1)
    dinv = dinv_ref[...]
    act = jnp.maximum(dinv * (acc + zc) + b_ref[...], 0.0)
    o_ref[...] = jnp.dot(act, w_ref[...], preferred_element_type=_f32) * dinv


def _tc_fused(acc, zp, dinv_col, b, W2):
    return pl.pallas_call(
        _tc_fused_body,
        grid=(GR, 2),
        in_specs=[
            pl.BlockSpec((2, BR, HH), lambda r, c: (0, r, 0)),
            pl.BlockSpec((2, BR, HH), lambda r, c: (0, r, 0)),
            pl.BlockSpec((BR, 1), lambda r, c: (r, 0)),
            pl.BlockSpec((1, H), lambda r, c: (0, 0)),
            pl.BlockSpec((H, HH), lambda r, c: (0, c)),
        ],
        out_specs=pl.BlockSpec((BR, HH), lambda r, c: (c * GR + r, 0)),
        out_shape=jax.ShapeDtypeStruct((2 * R, HH), _f32),
    )(acc.reshape(2, R, HH), zp.reshape(2, R, HH), dinv_col,
      b.reshape(1, H), W2)


def _tc_fused3(acc, zp, dinv_col, b, W3p):
    return pl.pallas_call(
        _tc_fused_body,
        grid=(GR, 1),
        in_specs=[
            pl.BlockSpec((2, BR, HH), lambda r, c: (0, r, 0)),
            pl.BlockSpec((2, BR, HH), lambda r, c: (0, r, 0)),
            pl.BlockSpec((BR, 1), lambda r, c: (r, 0)),
            pl.BlockSpec((1, H), lambda r, c: (0, 0)),
            pl.BlockSpec((H, HH), lambda r, c: (0, 0)),
        ],
        out_specs=pl.BlockSpec((BR, HH), lambda r, c: (r, 0)),
        out_shape=jax.ShapeDtypeStruct((R, HH), _f32),
    )(acc.reshape(2, R, HH), zp.reshape(2, R, HH), dinv_col,
      b.reshape(1, H), W3p)


def _tc_final_body(a_ref, z_ref, dinv_ref, b_ref, o_ref):
    sgt = dinv_ref[...] * (a_ref[0] + a_ref[1] + z_ref[...]) + b_ref[...]
    col = lax.broadcasted_iota(_i32, (BR, HH), 1)
    sm = jnp.where(col < C, sgt, -1e30)
    m = jnp.max(sm, axis=1, keepdims=True)
    e = jnp.where(col < C, jnp.exp(sm - m), 0.0)
    lse = jnp.log(jnp.sum(e, axis=1, keepdims=True))
    o_ref[...] = sm - m - lse


def _tc_final(acc3, z3p, dinv_col, b3p):
    return pl.pallas_call(
        _tc_final_body,
        grid=(GR,),
        in_specs=[
            pl.BlockSpec((2, BR, HH), lambda r: (0, r, 0)),
            pl.BlockSpec((BR, HH), lambda r: (r, 0)),
            pl.BlockSpec((BR, 1), lambda r: (r, 0)),
            pl.BlockSpec((1, HH), lambda r: (0, 0)),
        ],
        out_specs=pl.BlockSpec((BR, HH), lambda r: (r, 0)),
        out_shape=jax.ShapeDtypeStruct((R, HH), _f32),
    )(acc3.reshape(2, R, HH), z3p, dinv_col, b3p)


def _slab(v, nch, pad_val):
    per = v.shape[0] // NT
    v = v.reshape(NT, per)
    v = jnp.pad(v, ((0, 0), (0, nch * CH - per)), constant_values=pad_val)
    return v.reshape(NT, nch, CH)


def kernel(x, old_edge_index, new_edges, edge_probs, W1, b1, W2, b2, W3, b3):
    osrc = _slab(old_edge_index[0], NCH_OLD, N)
    odst = _slab(old_edge_index[1], NCH_OLD, N)
    nsrc = _slab(new_edges[0], NCH_NEW, N)
    ndst = _slab(new_edges[1], NCH_NEW, N)
    pslab = _slab(edge_probs, NCH_NEW, 0.0)

    x_pad = jnp.pad(x, ((0, R - N), (0, 0)))
    W3p = jnp.pad(W3, ((0, 0), (0, HH - C)))
    b3p = jnp.pad(b3, (0, HH - C)).reshape(1, HH)

    degp, wslab = _deg_w_kernel(odst, ndst, pslab)
    dinv_col = _tc_dinv(degp).reshape(R, 1)

    z1 = _tc_l1(x_pad, W1, dinv_col)
    a1 = _layer_half_kernel(z1, osrc, odst, nsrc, ndst, wslab)
    z2 = _tc_fused(a1, z1, dinv_col, b1, W2)
    a2 = _layer_half_kernel(z2, osrc, odst, nsrc, ndst, wslab)
    z3 = _tc_fused3(a2, z2, dinv_col, b2, W3p)
    a3 = _l3_kernel(z3, osrc, odst, nsrc, ndst, wslab)
    lp = _tc_final(a3, z3, dinv_col, b3p)

    log_probs = lp[:N, :C]
    updated_edge_index = jnp.concatenate([old_edge_index, new_edges], axis=1)
    return (log_probs, updated_edge_index)


# speedup vs baseline: 6.5399x; 1.1828x over previous
import functools

import jax
import jax.numpy as jnp
from jax import lax
from jax.experimental import pallas as pl
from jax.experimental.pallas import tpu as pltpu
from jax.experimental.pallas import tpu_sc as plsc

N = 10000
R = 10240
DF = 128
H = 256
HH = 128
C = 10
CP = 16
E_OLD = 320000
E_NEW = 156000
NT = 16
NCH_OLD = 160
NCH_NEW = 80
WG = 8
CH = 128
STRIPE = R // NT
BR = 512
GR = R // BR

_mesh = plsc.VectorSubcoreMesh(core_axis_name="c", subcore_axis_name="s")
_f32 = jnp.float32
_i32 = jnp.int32


def _zero_gbuf(gbuf, nrow, ncol16):
    def body(i, _):
        for k in range(ncol16):
            gbuf[i, pl.ds(k * 16, 16)] = jnp.zeros((16,), _f32)
        return 0
    lax.fori_loop(0, nrow, body, 0)


@functools.partial(
    pl.kernel,
    out_type=[
        jax.ShapeDtypeStruct((2, R), _f32),
        jax.ShapeDtypeStruct((NT, NCH_NEW, CH), _f32),
    ],
    mesh=_mesh,
    scratch_types=[
        pltpu.VMEM((NCH_OLD, CH), _i32),
        pltpu.VMEM((NCH_NEW, CH), _i32),
        pltpu.VMEM((NCH_NEW, CH), _f32),
        pltpu.VMEM((CH,), _f32),
        pltpu.VMEM((CH,), _f32),
        pltpu.VMEM((STRIPE,), _f32),
        pltpu.VMEM_SHARED((R,), _f32),
    ],
)
def _deg_w_kernel(odst_h, ndst_h, probs_h, degp_h, w_h,
                  odst, ndst, pslab, ones_v, wbuf, zbuf, deg_sp):
    c = lax.axis_index("c")
    s = lax.axis_index("s")

    def z16(i, _):
        zbuf[pl.ds(i * 16, 16)] = jnp.zeros((16,), _f32)
        return 0
    lax.fori_loop(0, STRIPE // 16, z16, 0)
    pltpu.sync_copy(zbuf, deg_sp.at[pl.ds(s * STRIPE, STRIPE)])
    for k in range(CH // 16):
        ones_v[pl.ds(k * 16, 16)] = jnp.ones((16,), _f32)
    pltpu.sync_copy(odst_h.at[s], odst)
    pltpu.sync_copy(ndst_h.at[s], ndst)
    pltpu.sync_copy(probs_h.at[s], pslab)
    plsc.subcore_barrier()

    ostart = c * (NCH_OLD // 2)

    def old_body(j, _):
        pltpu.sync_copy(ones_v, deg_sp.at[odst.at[j]], add=True)
        return 0
    lax.fori_loop(ostart, ostart + NCH_OLD // 2, old_body, 0)

    nstart = c * (NCH_NEW // 2)

    def new_body(j, _):
        for k in range(CH // 16):
            p = pslab[j, pl.ds(k * 16, 16)]
            wbuf[pl.ds(k * 16, 16)] = 1.0 / (1.0 + jnp.exp(-p))
        pltpu.sync_copy(wbuf, w_h.at[s, j])
        pltpu.sync_copy(wbuf, deg_sp.at[ndst.at[j]], add=True)
        return 0
    lax.fori_loop(nstart, nstart + NCH_NEW // 2, new_body, 0)

    plsc.subcore_barrier()
    pltpu.sync_copy(deg_sp.at[pl.ds(s * STRIPE, STRIPE)],
                    degp_h.at[c, pl.ds(s * STRIPE, STRIPE)])


@functools.partial(
    pl.kernel,
    out_type=jax.ShapeDtypeStruct((2 * R, HH), _f32),
    mesh=_mesh,
    scratch_types=[
        pltpu.VMEM((WG, CH), _i32),
        pltpu.VMEM((WG, CH), _i32),
        pltpu.VMEM((WG, CH), _f32),
        pltpu.VMEM((CH, HH), _f32),
        pltpu.VMEM((CH, HH), _f32),
        pltpu.VMEM_SHARED((R, HH), _f32),
        pltpu.SemaphoreType.DMA,
        pltpu.SemaphoreType.DMA,
        pltpu.SemaphoreType.DMA,
    ],
)
def _layer_half_kernel(z_h, osrc_h, odst_h, nsrc_h, ndst_h, w_h, acc_h,
                       srcw, dstw, ww, gb0, gb1, out_sp, gs0, gs1, ssem):
    c = lax.axis_index("c")
    s = lax.axis_index("s")
    off = c * R
    gbufs = (gb0, gb1)
    gsems = (gs0, gs1)

    _zero_gbuf(gb0, CH, HH // 16)
    for t in range(STRIPE // CH):
        pltpu.sync_copy(gb0, out_sp.at[pl.ds(s * STRIPE + t * CH, CH)])
    plsc.subcore_barrier()

    def drain_scatter():
        pltpu.make_async_copy(z_h.at[pl.ds(0, CH)], gb0, ssem).wait()

    def add_off(_, __):
        def body(j, ___):
            for k in range(CH // 16):
                srcw[j, pl.ds(k * 16, 16)] = srcw[j, pl.ds(k * 16, 16)] + off
            return 0
        lax.fori_loop(0, WG, body, 0)

    def pipelined_window(scale):
        g = [None] * WG
        g[0] = pltpu.async_copy(z_h.at[srcw.at[0]], gbufs[0], gsems[0])
        for j in range(WG):
            if j + 1 < WG:
                if j >= 1:
                    drain_scatter()
                g[j + 1] = pltpu.async_copy(
                    z_h.at[srcw.at[j + 1]], gbufs[(j + 1) % 2],
                    gsems[(j + 1) % 2])
            g[j].wait()
            if scale is not None:
                scale(j, gbufs[j % 2])
            pltpu.async_copy(gbufs[j % 2], out_sp.at[dstw.at[j]], ssem,
                             add=True)
        drain_scatter()
        drain_scatter()

    def old_win(wi, _):
        pltpu.sync_copy(osrc_h.at[s, pl.ds(wi * WG, WG)], srcw)
        pltpu.sync_copy(odst_h.at[s, pl.ds(wi * WG, WG)], dstw)
        add_off(None, None)
        pipelined_window(None)
        return 0
    lax.fori_loop(0, NCH_OLD // WG, old_win, 0)

    def scale_new(j, gbuf):
        def rowm(g, ___):
            wv = ww[j, pl.ds(g * 16, 16)]
            for t in range(16):
                i = g * 16 + t
                sw = wv[t]
                for k in range(HH // 16):
                    gbuf[i, pl.ds(k * 16, 16)] = (
                        gbuf[i, pl.ds(k * 16, 16)] * sw)
            return 0
        lax.fori_loop(0, CH // 16, rowm, 0)

    def new_win(wi, _):
        pltpu.sync_copy(nsrc_h.at[s, pl.ds(wi * WG, WG)], srcw)
        pltpu.sync_copy(ndst_h.at[s, pl.ds(wi * WG, WG)], dstw)
        pltpu.sync_copy(w_h.at[s, pl.ds(wi * WG, WG)], ww)
        add_off(None, None)
        pipelined_window(scale_new)
        return 0
    lax.fori_loop(0, NCH_NEW // WG, new_win, 0)

    plsc.subcore_barrier()
    pltpu.sync_copy(out_sp.at[pl.ds(s * STRIPE, STRIPE)],
                    acc_h.at[pl.ds(off + s * STRIPE, STRIPE)])


@functools.partial(
    pl.kernel,
    out_type=jax.ShapeDtypeStruct((2 * R, HH), _f32),
    mesh=_mesh,
    scratch_types=[
        pltpu.VMEM((WG, CH), _i32),
        pltpu.VMEM((WG, CH), _i32),
        pltpu.VMEM((WG, CH), _f32),
        pltpu.VMEM((CH, HH), _f32),
        pltpu.VMEM((CH, HH), _f32),
        pltpu.VMEM_SHARED((R, HH), _f32),
        pltpu.SemaphoreType.DMA,
        pltpu.SemaphoreType.DMA,
        pltpu.SemaphoreType.DMA,
    ],
)
def _l3_kernel(z_h, osrc_h, odst_h, nsrc_h, ndst_h, w_h, acc_h,
               srcw, dstw, ww, gb0, gb1, out_sp, gs0, gs1, ssem):
    c = lax.axis_index("c")
    s = lax.axis_index("s")
    off = c * R
    gbufs = (gb0, gb1)
    gsems = (gs0, gs1)

    _zero_gbuf(gb0, CH, HH // 16)
    for t in range(STRIPE // CH):
        pltpu.sync_copy(gb0, out_sp.at[pl.ds(s * STRIPE + t * CH, CH)])
    plsc.subcore_barrier()

    def drain_scatter():
        pltpu.make_async_copy(z_h.at[pl.ds(0, CH)], gb0, ssem).wait()

    def pipelined_window(scale):
        g = [None] * WG
        g[0] = pltpu.async_copy(z_h.at[srcw.at[0]], gbufs[0], gsems[0])
        for j in range(WG):
            if j + 1 < WG:
                if j >= 1:
                    drain_scatter()
                g[j + 1] = pltpu.async_copy(
                    z_h.at[srcw.at[j + 1]], gbufs[(j + 1) % 2],
                    gsems[(j + 1) % 2])
            g[j].wait()
            if scale is not None:
                scale(j, gbufs[j % 2])
            pltpu.async_copy(gbufs[j % 2], out_sp.at[dstw.at[j]], ssem,
                             add=True)
        drain_scatter()
        drain_scatter()

    ow0 = c * (NCH_OLD // WG // 2)

    def old_win(wi, _):
        pltpu.sync_copy(osrc_h.at[s, pl.ds(wi * WG, WG)], srcw)
        pltpu.sync_copy(odst_h.at[s, pl.ds(wi * WG, WG)], dstw)
        pipelined_window(None)
        return 0
    lax.fori_loop(ow0, ow0 + NCH_OLD // WG // 2, old_win, 0)

    nw0 = c * (NCH_NEW // WG // 2)

    def scale_new(j, gbuf):
        def rowm(g, ___):
            wv = ww[j, pl.ds(g * 16, 16)]
            for t in range(16):
                i = g * 16 + t
                gbuf[i, pl.ds(0, 16)] = gbuf[i, pl.ds(0, 16)] * wv[t]
            return 0
        lax.fori_loop(0, CH // 16, rowm, 0)

    def new_win(wi, _):
        pltpu.sync_copy(nsrc_h.at[s, pl.ds(wi * WG, WG)], srcw)
        pltpu.sync_copy(ndst_h.at[s, pl.ds(wi * WG, WG)], dstw)
        pltpu.sync_copy(w_h.at[s, pl.ds(wi * WG, WG)], ww)
        pipelined_window(scale_new)
        return 0
    lax.fori_loop(nw0, nw0 + NCH_NEW // WG // 2, new_win, 0)

    plsc.subcore_barrier()
    pltpu.sync_copy(out_sp.at[pl.ds(s * STRIPE, STRIPE)],
                    acc_h.at[pl.ds(off + s * STRIPE, STRIPE)])


def _tc_dinv_body(degp_ref, dinv_ref):
    d = degp_ref[0] + degp_ref[1] + 1.0
    row = lax.broadcasted_iota(_i32, (R // 128, 128), 0)
    col = lax.broadcasted_iota(_i32, (R // 128, 128), 1)
    valid = (row * 128 + col) < N
    dinv_ref[...] = jnp.where(valid, lax.rsqrt(d), 0.0)


def _tc_dinv(degp):
    return pl.pallas_call(
        _tc_dinv_body,
        out_shape=jax.ShapeDtypeStruct((R // 128, 128), _f32),
    )(degp.reshape(2, R // 128, 128))


def _tc_l1_body(x_ref, w_ref, dinv_ref, o_ref):
    o_ref[...] = jnp.dot(x_ref[...], w_ref[...],
                         preferred_element_type=_f32) * dinv_ref[...]


def _tc_l1(x_pad, W1, dinv_col):
    return pl.pallas_call(
        _tc_l1_body,
        grid=(GR, 2),
        in_specs=[
            pl.BlockSpec((BR, DF), lambda r, c: (r, 0)),
            pl.BlockSpec((DF, HH), lambda r, c: (0, c)),
            pl.BlockSpec((BR, 1), lambda r, c: (r, 0)),
        ],
        out_specs=pl.BlockSpec((BR, HH), lambda r, c: (c * GR + r, 0)),
        out_shape=jax.ShapeDtypeStruct((2 * R, HH), _f32),
    )(x_pad, W1, dinv_col)


def _tc_fused_body(a_ref, z_ref, dinv_ref, b_ref, w_ref, o_ref):
    acc = jnp.concatenate([a_ref[0], a_ref[1]], axis=1)
    zc = jnp.concatenate([z_ref[0], z_ref[1]], axis=1)
    dinv = dinv_ref[...]
    act = jnp.maximum(dinv * (acc + zc) + b_ref[...], 0.0)
    o_ref[...] = jnp.dot(act, w_ref[...], preferred_element_type=_f32) * dinv


def _tc_fused(acc, zp, dinv_col, b, W2):
    return pl.pallas_call(
        _tc_fused_body,
        grid=(GR, 2),
        in_specs=[
            pl.BlockSpec((2, BR, HH), lambda r, c: (0, r, 0)),
            pl.BlockSpec((2, BR, HH), lambda r, c: (0, r, 0)),
            pl.BlockSpec((BR, 1), lambda r, c: (r, 0)),
            pl.BlockSpec((1, H), lambda r, c: (0, 0)),
            pl.BlockSpec((H, HH), lambda r, c: (0, c)),
        ],
        out_specs=pl.BlockSpec((BR, HH), lambda r, c: (c * GR + r, 0)),
        out_shape=jax.ShapeDtypeStruct((2 * R, HH), _f32),
    )(acc.reshape(2, R, HH), zp.reshape(2, R, HH), dinv_col,
      b.reshape(1, H), W2)


def _tc_fused3(acc, zp, dinv_col, b, W3p):
    return pl.pallas_call(
        _tc_fused_body,
        grid=(GR, 1),
        in_specs=[
            pl.BlockSpec((2, BR, HH), lambda r, c: (0, r, 0)),
            pl.BlockSpec((2, BR, HH), lambda r, c: (0, r, 0)),
            pl.BlockSpec((BR, 1), lambda r, c: (r, 0)),
            pl.BlockSpec((1, H), lambda r, c: (0, 0)),
            pl.BlockSpec((H, HH), lambda r, c: (0, 0)),
        ],
        out_specs=pl.BlockSpec((BR, HH), lambda r, c: (r, 0)),
        out_shape=jax.ShapeDtypeStruct((R, HH), _f32),
    )(acc.reshape(2, R, HH), zp.reshape(2, R, HH), dinv_col,
      b.reshape(1, H), W3p)


def _tc_final_body(a_ref, z_ref, dinv_ref, b_ref, o_ref):
    sgt = dinv_ref[...] * (a_ref[0] + a_ref[1] + z_ref[...]) + b_ref[...]
    col = lax.broadcasted_iota(_i32, (BR, HH), 1)
    sm = jnp.where(col < C, sgt, -1e30)
    m = jnp.max(sm, axis=1, keepdims=True)
    e = jnp.where(col < C, jnp.exp(sm - m), 0.0)
    lse = jnp.log(jnp.sum(e, axis=1, keepdims=True))
    o_ref[...] = sm - m - lse


def _tc_final(acc3, z3p, dinv_col, b3p):
    return pl.pallas_call(
        _tc_final_body,
        grid=(GR,),
        in_specs=[
            pl.BlockSpec((2, BR, HH), lambda r: (0, r, 0)),
            pl.BlockSpec((BR, HH), lambda r: (r, 0)),
            pl.BlockSpec((BR, 1), lambda r: (r, 0)),
            pl.BlockSpec((1, HH), lambda r: (0, 0)),
        ],
        out_specs=pl.BlockSpec((BR, HH), lambda r: (r, 0)),
        out_shape=jax.ShapeDtypeStruct((R, HH), _f32),
    )(acc3.reshape(2, R, HH), z3p, dinv_col, b3p)


def _slab(v, nch, pad_val):
    per = v.shape[0] // NT
    v = v.reshape(NT, per)
    v = jnp.pad(v, ((0, 0), (0, nch * CH - per)), constant_values=pad_val)
    return v.reshape(NT, nch, CH)


def kernel(x, old_edge_index, new_edges, edge_probs, W1, b1, W2, b2, W3, b3):
    osrc = _slab(old_edge_index[0], NCH_OLD, N)
    odst = _slab(old_edge_index[1], NCH_OLD, N)
    nsrc = _slab(new_edges[0], NCH_NEW, N)
    ndst = _slab(new_edges[1], NCH_NEW, N)
    pslab = _slab(edge_probs, NCH_NEW, 0.0)

    x_pad = jnp.pad(x, ((0, R - N), (0, 0)))
    W3p = jnp.pad(W3, ((0, 0), (0, HH - C)))
    b3p = jnp.pad(b3, (0, HH - C)).reshape(1, HH)

    degp, wslab = _deg_w_kernel(odst, ndst, pslab)
    dinv_col = _tc_dinv(degp).reshape(R, 1)

    z1 = _tc_l1(x_pad, W1, dinv_col)
    a1 = _layer_half_kernel(z1, osrc, odst, nsrc, ndst, wslab)
    z2 = _tc_fused(a1, z1, dinv_col, b1, W2)
    a2 = _layer_half_kernel(z2, osrc, odst, nsrc, ndst, wslab)
    z3 = _tc_fused3(a2, z2, dinv_col, b2, W3p)
    a3 = _l3_kernel(z3, osrc, odst, nsrc, ndst, wslab)
    lp = _tc_final(a3, z3, dinv_col, b3p)

    log_probs = lp[:N, :C]
    updated_edge_index = jnp.concatenate([old_edge_index, new_edges], axis=1)
    return (log_probs, updated_edge_index)


# speedup vs baseline: 13.1594x; 2.0122x over previous
import functools

import jax
import jax.numpy as jnp
from jax import lax
from jax.experimental import pallas as pl
from jax.experimental.pallas import tpu as pltpu
from jax.experimental.pallas import tpu_sc as plsc

N = 10000
R = 10240
DF = 128
H = 256
HH = 128
C = 10
CP = 16
E_OLD = 320000
E_NEW = 156000
NT = 16
NCH_OLD = 160
NCH_NEW = 80
WG = 8
CH = 128
STRIPE = R // NT
BR = 512
GR = R // BR

_mesh = plsc.VectorSubcoreMesh(core_axis_name="c", subcore_axis_name="s")
_f32 = jnp.float32
_i32 = jnp.int32


def _zero_gbuf(gbuf, nrow, ncol16):
    def body(i, _):
        for k in range(ncol16):
            gbuf[i, pl.ds(k * 16, 16)] = jnp.zeros((16,), _f32)
        return 0
    lax.fori_loop(0, nrow, body, 0)


@functools.partial(
    pl.kernel,
    out_type=[
        jax.ShapeDtypeStruct((2, R), _f32),
        jax.ShapeDtypeStruct((NT, NCH_NEW, CH), _f32),
    ],
    mesh=_mesh,
    scratch_types=[
        pltpu.VMEM((NCH_OLD, CH), _i32),
        pltpu.VMEM((NCH_NEW, CH), _i32),
        pltpu.VMEM((NCH_NEW, CH), _f32),
        pltpu.VMEM((CH,), _f32),
        pltpu.VMEM((CH,), _f32),
        pltpu.VMEM((STRIPE,), _f32),
        pltpu.VMEM_SHARED((R,), _f32),
    ],
)
def _deg_w_kernel(odst_h, ndst_h, probs_h, degp_h, w_h,
                  odst, ndst, pslab, ones_v, wbuf, zbuf, deg_sp):
    c = lax.axis_index("c")
    s = lax.axis_index("s")

    def z16(i, _):
        zbuf[pl.ds(i * 16, 16)] = jnp.zeros((16,), _f32)
        return 0
    lax.fori_loop(0, STRIPE // 16, z16, 0)
    pltpu.sync_copy(zbuf, deg_sp.at[pl.ds(s * STRIPE, STRIPE)])
    for k in range(CH // 16):
        ones_v[pl.ds(k * 16, 16)] = jnp.ones((16,), _f32)
    pltpu.sync_copy(odst_h.at[s], odst)
    pltpu.sync_copy(ndst_h.at[s], ndst)
    pltpu.sync_copy(probs_h.at[s], pslab)
    plsc.subcore_barrier()

    ostart = c * (NCH_OLD // 2)

    def old_body(j, _):
        pltpu.sync_copy(ones_v, deg_sp.at[odst.at[j]], add=True)
        return 0
    lax.fori_loop(ostart, ostart + NCH_OLD // 2, old_body, 0)

    nstart = c * (NCH_NEW // 2)

    def new_body(j, _):
        for k in range(CH // 16):
            p = pslab[j, pl.ds(k * 16, 16)]
            wbuf[pl.ds(k * 16, 16)] = 1.0 / (1.0 + jnp.exp(-p))
        pltpu.sync_copy(wbuf, w_h.at[s, j])
        pltpu.sync_copy(wbuf, deg_sp.at[ndst.at[j]], add=True)
        return 0
    lax.fori_loop(nstart, nstart + NCH_NEW // 2, new_body, 0)

    plsc.subcore_barrier()
    pltpu.sync_copy(deg_sp.at[pl.ds(s * STRIPE, STRIPE)],
                    degp_h.at[c, pl.ds(s * STRIPE, STRIPE)])


@functools.partial(
    pl.kernel,
    out_type=jax.ShapeDtypeStruct((2 * R, HH), _f32),
    mesh=_mesh,
    scratch_types=[
        pltpu.VMEM((WG, CH), _i32),
        pltpu.VMEM((WG, CH), _i32),
        pltpu.VMEM((WG, CH), _f32),
        pltpu.VMEM((CH, HH), _f32),
        pltpu.VMEM((CH, HH), _f32),
        pltpu.VMEM_SHARED((R, HH), _f32),
        pltpu.SemaphoreType.DMA,
        pltpu.SemaphoreType.DMA,
        pltpu.SemaphoreType.DMA,
    ],
)
def _layer_half_kernel(z_h, osrc_h, odst_h, nsrc_h, ndst_h, w_h, acc_h,
                       srcw, dstw, ww, gb0, gb1, out_sp, gs0, gs1, ssem):
    c = lax.axis_index("c")
    s = lax.axis_index("s")
    off = c * R
    gbufs = (gb0, gb1)
    gsems = (gs0, gs1)

    _zero_gbuf(gb0, CH, HH // 16)
    for t in range(STRIPE // CH):
        pltpu.sync_copy(gb0, out_sp.at[pl.ds(s * STRIPE + t * CH, CH)])
    plsc.subcore_barrier()

    def drain_scatter():
        pltpu.make_async_copy(z_h.at[pl.ds(0, CH)], gb0, ssem).wait()

    def add_off(_, __):
        def body(j, ___):
            for k in range(CH // 16):
                srcw[j, pl.ds(k * 16, 16)] = srcw[j, pl.ds(k * 16, 16)] + off
            return 0
        lax.fori_loop(0, WG, body, 0)

    def pipelined_window(scale):
        g = [None] * WG
        g[0] = pltpu.async_copy(z_h.at[srcw.at[0]], gbufs[0], gsems[0])
        for j in range(WG):
            if j + 1 < WG:
                if j >= 1:
                    drain_scatter()
                g[j + 1] = pltpu.async_copy(
                    z_h.at[srcw.at[j + 1]], gbufs[(j + 1) % 2],
                    gsems[(j + 1) % 2])
            g[j].wait()
            if scale is not None:
                scale(j, gbufs[j % 2])
            pltpu.async_copy(gbufs[j % 2], out_sp.at[dstw.at[j]], ssem,
                             add=True)
        drain_scatter()
        drain_scatter()

    def old_win(wi, _):
        pltpu.sync_copy(osrc_h.at[s, pl.ds(wi * WG, WG)], srcw)
        pltpu.sync_copy(odst_h.at[s, pl.ds(wi * WG, WG)], dstw)
        add_off(None, None)
        pipelined_window(None)
        return 0
    lax.fori_loop(0, NCH_OLD // WG, old_win, 0)

    def scale_new(j, gbuf):
        def rowm(g, ___):
            wv = ww[j, pl.ds(g * 16, 16)]
            for t in range(16):
                i = g * 16 + t
                sw = wv[t]
                for k in range(HH // 16):
                    gbuf[i, pl.ds(k * 16, 16)] = (
                        gbuf[i, pl.ds(k * 16, 16)] * sw)
            return 0
        lax.fori_loop(0, CH // 16, rowm, 0)

    def new_win(wi, _):
        pltpu.sync_copy(nsrc_h.at[s, pl.ds(wi * WG, WG)], srcw)
        pltpu.sync_copy(ndst_h.at[s, pl.ds(wi * WG, WG)], dstw)
        pltpu.sync_copy(w_h.at[s, pl.ds(wi * WG, WG)], ww)
        add_off(None, None)
        pipelined_window(scale_new)
        return 0
    lax.fori_loop(0, NCH_NEW // WG, new_win, 0)

    plsc.subcore_barrier()
    pltpu.sync_copy(out_sp.at[pl.ds(s * STRIPE, STRIPE)],
                    acc_h.at[pl.ds(off + s * STRIPE, STRIPE)])


@functools.partial(
    pl.kernel,
    out_type=jax.ShapeDtypeStruct((2 * R, HH), _f32),
    mesh=_mesh,
    scratch_types=[
        pltpu.VMEM((WG, CH), _i32),
        pltpu.VMEM((WG, CH), _i32),
        pltpu.VMEM((WG, CH), _f32),
        pltpu.VMEM((CH, HH), _f32),
        pltpu.VMEM((CH, HH), _f32),
        pltpu.VMEM_SHARED((R, HH), _f32),
        pltpu.SemaphoreType.DMA,
        pltpu.SemaphoreType.DMA,
        pltpu.SemaphoreType.DMA,
    ],
)
def _l3_kernel(z_h, osrc_h, odst_h, nsrc_h, ndst_h, w_h, acc_h,
               srcw, dstw, ww, gb0, gb1, out_sp, gs0, gs1, ssem):
    c = lax.axis_index("c")
    s = lax.axis_index("s")
    off = c * R
    gbufs = (gb0, gb1)
    gsems = (gs0, gs1)

    _zero_gbuf(gb0, CH, HH // 16)
    for t in range(STRIPE // CH):
        pltpu.sync_copy(gb0, out_sp.at[pl.ds(s * STRIPE + t * CH, CH)])
    plsc.subcore_barrier()

    def drain_scatter():
        pltpu.make_async_copy(z_h.at[pl.ds(0, CH)], gb0, ssem).wait()

    def pipelined_window(scale):
        g = [None] * WG
        g[0] = pltpu.async_copy(z_h.at[srcw.at[0]], gbufs[0], gsems[0])
        for j in range(WG):
            if j + 1 < WG:
                if j >= 1:
                    drain_scatter()
                g[j + 1] = pltpu.async_copy(
                    z_h.at[srcw.at[j + 1]], gbufs[(j + 1) % 2],
                    gsems[(j + 1) % 2])
            g[j].wait()
            if scale is not None:
                scale(j, gbufs[j % 2])
            pltpu.async_copy(gbufs[j % 2], out_sp.at[dstw.at[j]], ssem,
                             add=True)
        drain_scatter()
        drain_scatter()

    ow0 = c * (NCH_OLD // WG // 2)

    def old_win(wi, _):
        pltpu.sync_copy(osrc_h.at[s, pl.ds(wi * WG, WG)], srcw)
        pltpu.sync_copy(odst_h.at[s, pl.ds(wi * WG, WG)], dstw)
        pipelined_window(None)
        return 0
    lax.fori_loop(ow0, ow0 + NCH_OLD // WG // 2, old_win, 0)

    nw0 = c * (NCH_NEW // WG // 2)

    def scale_new(j, gbuf):
        def rowm(g, ___):
            wv = ww[j, pl.ds(g * 16, 16)]
            for t in range(16):
                i = g * 16 + t
                gbuf[i, pl.ds(0, 16)] = gbuf[i, pl.ds(0, 16)] * wv[t]
            return 0
        lax.fori_loop(0, CH // 16, rowm, 0)

    def new_win(wi, _):
        pltpu.sync_copy(nsrc_h.at[s, pl.ds(wi * WG, WG)], srcw)
        pltpu.sync_copy(ndst_h.at[s, pl.ds(wi * WG, WG)], dstw)
        pltpu.sync_copy(w_h.at[s, pl.ds(wi * WG, WG)], ww)
        pipelined_window(scale_new)
        return 0
    lax.fori_loop(nw0, nw0 + NCH_NEW // WG // 2, new_win, 0)

    plsc.subcore_barrier()
    pltpu.sync_copy(out_sp.at[pl.ds(s * STRIPE, STRIPE)],
                    acc_h.at[pl.ds(off + s * STRIPE, STRIPE)])


def _tc_dinv_body(degp_ref, dinv_ref):
    d = degp_ref[0] + degp_ref[1] + 1.0
    row = lax.broadcasted_iota(_i32, (R // 128, 128), 0)
    col = lax.broadcasted_iota(_i32, (R // 128, 128), 1)
    valid = (row * 128 + col) < N
    dinv_ref[...] = jnp.where(valid, lax.rsqrt(d), 0.0)


def _tc_dinv(degp):
    return pl.pallas_call(
        _tc_dinv_body,
        out_shape=jax.ShapeDtypeStruct((R // 128, 128), _f32),
    )(degp.reshape(2, R // 128, 128))


def _tc_l1_body(x_ref, w_ref, dinv_ref, o_ref):
    o_ref[...] = jnp.dot(x_ref[...], w_ref[...],
                         preferred_element_type=_f32) * dinv_ref[...]


def _tc_l1(x_pad, W1, dinv_col):
    return pl.pallas_call(
        _tc_l1_body,
        grid=(GR, 2),
        in_specs=[
            pl.BlockSpec((BR, DF), lambda r, c: (r, 0)),
            pl.BlockSpec((DF, HH), lambda r, c: (0, c)),
            pl.BlockSpec((BR, 1), lambda r, c: (r, 0)),
        ],
        out_specs=pl.BlockSpec((BR, HH), lambda r, c: (c * GR + r, 0)),
        out_shape=jax.ShapeDtypeStruct((2 * R, HH), _f32),
    )(x_pad, W1, dinv_col)


def _tc_fused_body(a_ref, z_ref, dinv_ref, b_ref, w_ref, o_ref):
    acc = jnp.concatenate([a_ref[0], a_ref[1]], axis=1)
    zc = jnp.concatenate([z_ref[0], z_ref[1]], axis=1)
    dinv = dinv_ref[...]
    act = jnp.maximum(dinv * (acc + zc) + b_ref[...], 0.0)
    o_ref[...] = jnp.dot(act, w_ref[...], preferred_element_type=_f32) * dinv


def _tc_fused(acc, zp, dinv_col, b, W2):
    return pl.pallas_call(
        _tc_fused_body,
        grid=(GR, 2),
        in_specs=[
            pl.BlockSpec((2, BR, HH), lambda r, c: (0, r, 0)),
            pl.BlockSpec((2, BR, HH), lambda r, c: (0, r, 0)),
            pl.BlockSpec((BR, 1), lambda r, c: (r, 0)),
            pl.BlockSpec((1, H), lambda r, c: (0, 0)),
            pl.BlockSpec((H, HH), lambda r, c: (0, c)),
        ],
        out_specs=pl.BlockSpec((BR, HH), lambda r, c: (c * GR + r, 0)),
        out_shape=jax.ShapeDtypeStruct((2 * R, HH), _f32),
    )(acc.reshape(2, R, HH), zp.reshape(2, R, HH), dinv_col,
      b.reshape(1, H), W2)


def _tc_fused3(acc, zp, dinv_col, b, W3p):
    return pl.pallas_call(
        _tc_fused_body,
        grid=(GR, 1),
        in_specs=[
            pl.BlockSpec((2, BR, HH), lambda r, c: (0, r, 0)),
            pl.BlockSpec((2, BR, HH), lambda r, c: (0, r, 0)),
            pl.BlockSpec((BR, 1), lambda r, c: (r, 0)),
            pl.BlockSpec((1, H), lambda r, c: (0, 0)),
            pl.BlockSpec((H, HH), lambda r, c: (0, 0)),
        ],
        out_specs=pl.BlockSpec((BR, HH), lambda r, c: (r, 0)),
        out_shape=jax.ShapeDtypeStruct((R, HH), _f32),
    )(acc.reshape(2, R, HH), zp.reshape(2, R, HH), dinv_col,
      b.reshape(1, H), W3p)


def _tc_final_body(a_ref, z_ref, dinv_ref, b_ref, o_ref):
    sgt = dinv_ref[...] * (a_ref[0] + a_ref[1] + z_ref[...]) + b_ref[...]
    col = lax.broadcasted_iota(_i32, (BR, HH), 1)
    sm = jnp.where(col < C, sgt, -1e30)
    m = jnp.max(sm, axis=1, keepdims=True)
    e = jnp.where(col < C, jnp.exp(sm - m), 0.0)
    lse = jnp.log(jnp.sum(e, axis=1, keepdims=True))
    o_ref[...] = sm - m - lse


def _tc_final(acc3, z3p, dinv_col, b3p):
    return pl.pallas_call(
        _tc_final_body,
        grid=(GR,),
        in_specs=[
            pl.BlockSpec((2, BR, HH), lambda r: (0, r, 0)),
            pl.BlockSpec((BR, HH), lambda r: (r, 0)),
            pl.BlockSpec((BR, 1), lambda r: (r, 0)),
            pl.BlockSpec((1, HH), lambda r: (0, 0)),
        ],
        out_specs=pl.BlockSpec((BR, HH), lambda r: (r, 0)),
        out_shape=jax.ShapeDtypeStruct((R, HH), _f32),
    )(acc3.reshape(2, R, HH), z3p, dinv_col, b3p)


def _slab(v, nch, pad_val):
    per = v.shape[0] // NT
    v = v.reshape(NT, per)
    v = jnp.pad(v, ((0, 0), (0, nch * CH - per)), constant_values=pad_val)
    return v.reshape(NT, nch, CH)


def _slab_idx(v, nch):
    per = v.shape[0] // NT
    npad = nch * CH - per
    padv = N + (jnp.arange(npad, dtype=v.dtype) % (R - N))
    v = v.reshape(NT, per)
    v = jnp.concatenate([v, jnp.broadcast_to(padv, (NT, npad))], axis=1)
    return v.reshape(NT, nch, CH)


def kernel(x, old_edge_index, new_edges, edge_probs, W1, b1, W2, b2, W3, b3):
    osrc = _slab_idx(old_edge_index[0], NCH_OLD)
    odst = _slab_idx(old_edge_index[1], NCH_OLD)
    nsrc = _slab_idx(new_edges[0], NCH_NEW)
    ndst = _slab_idx(new_edges[1], NCH_NEW)
    pslab = _slab(edge_probs, NCH_NEW, 0.0)

    x_pad = jnp.pad(x, ((0, R - N), (0, 0)))
    W3p = jnp.pad(W3, ((0, 0), (0, HH - C)))
    b3p = jnp.pad(b3, (0, HH - C)).reshape(1, HH)

    degp, wslab = _deg_w_kernel(odst, ndst, pslab)
    dinv_col = _tc_dinv(degp).reshape(R, 1)

    z1 = _tc_l1(x_pad, W1, dinv_col)
    a1 = _layer_half_kernel(z1, osrc, odst, nsrc, ndst, wslab)
    z2 = _tc_fused(a1, z1, dinv_col, b1, W2)
    a2 = _layer_half_kernel(z2, osrc, odst, nsrc, ndst, wslab)
    z3 = _tc_fused3(a2, z2, dinv_col, b2, W3p)
    a3 = _l3_kernel(z3, osrc, odst, nsrc, ndst, wslab)
    lp = _tc_final(a3, z3, dinv_col, b3p)

    log_probs = lp[:N, :C]
    updated_edge_index = jnp.concatenate([old_edge_index, new_edges], axis=1)
    return (log_probs, updated_edge_index)


# speedup vs baseline: 14.3498x; 1.0905x over previous
import functools

import jax
import jax.numpy as jnp
from jax import lax
from jax.experimental import pallas as pl
from jax.experimental.pallas import tpu as pltpu
from jax.experimental.pallas import tpu_sc as plsc

N = 10000
R = 10240
DF = 128
H = 256
HH = 128
C = 10
CP = 16
E_OLD = 320000
E_NEW = 156000
NT = 16
NCH_OLD = 160
NCH_NEW = 80
WG = 16
CH = 128
STRIPE = R // NT
BR = 512
GR = R // BR

_mesh = plsc.VectorSubcoreMesh(core_axis_name="c", subcore_axis_name="s")
_f32 = jnp.float32
_i32 = jnp.int32


def _zero_gbuf(gbuf, nrow, ncol16):
    def body(i, _):
        for k in range(ncol16):
            gbuf[i, pl.ds(k * 16, 16)] = jnp.zeros((16,), _f32)
        return 0
    lax.fori_loop(0, nrow, body, 0)


@functools.partial(
    pl.kernel,
    out_type=[
        jax.ShapeDtypeStruct((2, R), _f32),
        jax.ShapeDtypeStruct((NT, NCH_NEW, CH), _f32),
    ],
    mesh=_mesh,
    scratch_types=[
        pltpu.VMEM((NCH_OLD, CH), _i32),
        pltpu.VMEM((NCH_NEW, CH), _i32),
        pltpu.VMEM((NCH_NEW, CH), _f32),
        pltpu.VMEM((CH,), _f32),
        pltpu.VMEM((CH,), _f32),
        pltpu.VMEM((STRIPE,), _f32),
        pltpu.VMEM_SHARED((R,), _f32),
    ],
)
def _deg_w_kernel(odst_h, ndst_h, probs_h, degp_h, w_h,
                  odst, ndst, pslab, ones_v, wbuf, zbuf, deg_sp):
    c = lax.axis_index("c")
    s = lax.axis_index("s")

    def z16(i, _):
        zbuf[pl.ds(i * 16, 16)] = jnp.zeros((16,), _f32)
        return 0
    lax.fori_loop(0, STRIPE // 16, z16, 0)
    pltpu.sync_copy(zbuf, deg_sp.at[pl.ds(s * STRIPE, STRIPE)])
    for k in range(CH // 16):
        ones_v[pl.ds(k * 16, 16)] = jnp.ones((16,), _f32)
    pltpu.sync_copy(odst_h.at[s], odst)
    pltpu.sync_copy(ndst_h.at[s], ndst)
    pltpu.sync_copy(probs_h.at[s], pslab)
    plsc.subcore_barrier()

    ostart = c * (NCH_OLD // 2)

    def old_body(j, _):
        pltpu.sync_copy(ones_v, deg_sp.at[odst.at[j]], add=True)
        return 0
    lax.fori_loop(ostart, ostart + NCH_OLD // 2, old_body, 0)

    nstart = c * (NCH_NEW // 2)

    def new_body(j, _):
        for k in range(CH // 16):
            p = pslab[j, pl.ds(k * 16, 16)]
            wbuf[pl.ds(k * 16, 16)] = 1.0 / (1.0 + jnp.exp(-p))
        pltpu.sync_copy(wbuf, w_h.at[s, j])
        pltpu.sync_copy(wbuf, deg_sp.at[ndst.at[j]], add=True)
        return 0
    lax.fori_loop(nstart, nstart + NCH_NEW // 2, new_body, 0)

    plsc.subcore_barrier()
    pltpu.sync_copy(deg_sp.at[pl.ds(s * STRIPE, STRIPE)],
                    degp_h.at[c, pl.ds(s * STRIPE, STRIPE)])


@functools.partial(
    pl.kernel,
    out_type=jax.ShapeDtypeStruct((2 * R, HH), _f32),
    mesh=_mesh,
    scratch_types=[
        pltpu.VMEM((WG, CH), _i32),
        pltpu.VMEM((WG, CH), _i32),
        pltpu.VMEM((WG, CH), _f32),
        pltpu.VMEM((CH, HH), _f32),
        pltpu.VMEM((CH, HH), _f32),
        pltpu.VMEM_SHARED((R, HH), _f32),
        pltpu.SemaphoreType.DMA,
        pltpu.SemaphoreType.DMA,
        pltpu.SemaphoreType.DMA,
    ],
)
def _layer_half_kernel(z_h, osrc_h, odst_h, nsrc_h, ndst_h, w_h, acc_h,
                       srcw, dstw, ww, gb0, gb1, out_sp, gs0, gs1, ssem):
    c = lax.axis_index("c")
    s = lax.axis_index("s")
    off = c * R
    gbufs = (gb0, gb1)
    gsems = (gs0, gs1)

    _zero_gbuf(gb0, CH, HH // 16)
    for t in range(STRIPE // CH):
        pltpu.sync_copy(gb0, out_sp.at[pl.ds(s * STRIPE + t * CH, CH)])
    plsc.subcore_barrier()

    def drain_scatter():
        pltpu.make_async_copy(z_h.at[pl.ds(0, CH)], gb0, ssem).wait()

    def add_off(_, __):
        def body(j, ___):
            for k in range(CH // 16):
                srcw[j, pl.ds(k * 16, 16)] = srcw[j, pl.ds(k * 16, 16)] + off
            return 0
        lax.fori_loop(0, WG, body, 0)

    def pipelined_window(scale):
        g = [None] * WG
        g[0] = pltpu.async_copy(z_h.at[srcw.at[0]], gbufs[0], gsems[0])
        for j in range(WG):
            if j + 1 < WG:
                if j >= 1:
                    drain_scatter()
                g[j + 1] = pltpu.async_copy(
                    z_h.at[srcw.at[j + 1]], gbufs[(j + 1) % 2],
                    gsems[(j + 1) % 2])
            g[j].wait()
            if scale is not None:
                scale(j, gbufs[j % 2])
            pltpu.async_copy(gbufs[j % 2], out_sp.at[dstw.at[j]], ssem,
                             add=True)
        drain_scatter()
        drain_scatter()

    def old_win(wi, _):
        pltpu.sync_copy(osrc_h.at[s, pl.ds(wi * WG, WG)], srcw)
        pltpu.sync_copy(odst_h.at[s, pl.ds(wi * WG, WG)], dstw)
        add_off(None, None)
        pipelined_window(None)
        return 0
    lax.fori_loop(0, NCH_OLD // WG, old_win, 0)

    def scale_new(j, gbuf):
        def rowm(g, ___):
            wv = ww[j, pl.ds(g * 16, 16)]
            for t in range(16):
                i = g * 16 + t
                sw = wv[t]
                for k in range(HH // 16):
                    gbuf[i, pl.ds(k * 16, 16)] = (
                        gbuf[i, pl.ds(k * 16, 16)] * sw)
            return 0
        lax.fori_loop(0, CH // 16, rowm, 0)

    def new_win(wi, _):
        pltpu.sync_copy(nsrc_h.at[s, pl.ds(wi * WG, WG)], srcw)
        pltpu.sync_copy(ndst_h.at[s, pl.ds(wi * WG, WG)], dstw)
        pltpu.sync_copy(w_h.at[s, pl.ds(wi * WG, WG)], ww)
        add_off(None, None)
        pipelined_window(scale_new)
        return 0
    lax.fori_loop(0, NCH_NEW // WG, new_win, 0)

    plsc.subcore_barrier()
    pltpu.sync_copy(out_sp.at[pl.ds(s * STRIPE, STRIPE)],
                    acc_h.at[pl.ds(off + s * STRIPE, STRIPE)])


@functools.partial(
    pl.kernel,
    out_type=jax.ShapeDtypeStruct((2 * R, HH), _f32),
    mesh=_mesh,
    scratch_types=[
        pltpu.VMEM((WG, CH), _i32),
        pltpu.VMEM((WG, CH), _i32),
        pltpu.VMEM((WG, CH), _f32),
        pltpu.VMEM((CH, HH), _f32),
        pltpu.VMEM((CH, HH), _f32),
        pltpu.VMEM_SHARED((R, HH), _f32),
        pltpu.SemaphoreType.DMA,
        pltpu.SemaphoreType.DMA,
        pltpu.SemaphoreType.DMA,
    ],
)
def _l3_kernel(z_h, osrc_h, odst_h, nsrc_h, ndst_h, w_h, acc_h,
               srcw, dstw, ww, gb0, gb1, out_sp, gs0, gs1, ssem):
    c = lax.axis_index("c")
    s = lax.axis_index("s")
    off = c * R
    gbufs = (gb0, gb1)
    gsems = (gs0, gs1)

    _zero_gbuf(gb0, CH, HH // 16)
    for t in range(STRIPE // CH):
        pltpu.sync_copy(gb0, out_sp.at[pl.ds(s * STRIPE + t * CH, CH)])
    plsc.subcore_barrier()

    def drain_scatter():
        pltpu.make_async_copy(z_h.at[pl.ds(0, CH)], gb0, ssem).wait()

    def pipelined_window(scale):
        g = [None] * WG
        g[0] = pltpu.async_copy(z_h.at[srcw.at[0]], gbufs[0], gsems[0])
        for j in range(WG):
            if j + 1 < WG:
                if j >= 1:
                    drain_scatter()
                g[j + 1] = pltpu.async_copy(
                    z_h.at[srcw.at[j + 1]], gbufs[(j + 1) % 2],
                    gsems[(j + 1) % 2])
            g[j].wait()
            if scale is not None:
                scale(j, gbufs[j % 2])
            pltpu.async_copy(gbufs[j % 2], out_sp.at[dstw.at[j]], ssem,
                             add=True)
        drain_scatter()
        drain_scatter()

    ow0 = c * (NCH_OLD // WG // 2)

    def old_win(wi, _):
        pltpu.sync_copy(osrc_h.at[s, pl.ds(wi * WG, WG)], srcw)
        pltpu.sync_copy(odst_h.at[s, pl.ds(wi * WG, WG)], dstw)
        pipelined_window(None)
        return 0
    lax.fori_loop(ow0, ow0 + NCH_OLD // WG // 2, old_win, 0)

    nw0 = c * (NCH_NEW // WG // 2)

    def scale_new(j, gbuf):
        def rowm(g, ___):
            wv = ww[j, pl.ds(g * 16, 16)]
            for t in range(16):
                i = g * 16 + t
                gbuf[i, pl.ds(0, 16)] = gbuf[i, pl.ds(0, 16)] * wv[t]
            return 0
        lax.fori_loop(0, CH // 16, rowm, 0)

    def new_win(wi, _):
        pltpu.sync_copy(nsrc_h.at[s, pl.ds(wi * WG, WG)], srcw)
        pltpu.sync_copy(ndst_h.at[s, pl.ds(wi * WG, WG)], dstw)
        pltpu.sync_copy(w_h.at[s, pl.ds(wi * WG, WG)], ww)
        pipelined_window(scale_new)
        return 0
    lax.fori_loop(nw0, nw0 + NCH_NEW // WG // 2, new_win, 0)

    plsc.subcore_barrier()
    pltpu.sync_copy(out_sp.at[pl.ds(s * STRIPE, STRIPE)],
                    acc_h.at[pl.ds(off + s * STRIPE, STRIPE)])


def _tc_dinv_body(degp_ref, dinv_ref):
    d = degp_ref[0] + degp_ref[1] + 1.0
    row = lax.broadcasted_iota(_i32, (R // 128, 128), 0)
    col = lax.broadcasted_iota(_i32, (R // 128, 128), 1)
    valid = (row * 128 + col) < N
    dinv_ref[...] = jnp.where(valid, lax.rsqrt(d), 0.0)


def _tc_dinv(degp):
    return pl.pallas_call(
        _tc_dinv_body,
        out_shape=jax.ShapeDtypeStruct((R // 128, 128), _f32),
    )(degp.reshape(2, R // 128, 128))


def _tc_l1_body(x_ref, w_ref, dinv_ref, o_ref):
    o_ref[...] = jnp.dot(x_ref[...], w_ref[...],
                         preferred_element_type=_f32) * dinv_ref[...]


def _tc_l1(x_pad, W1, dinv_col):
    return pl.pallas_call(
        _tc_l1_body,
        grid=(GR, 2),
        in_specs=[
            pl.BlockSpec((BR, DF), lambda r, c: (r, 0)),
            pl.BlockSpec((DF, HH), lambda r, c: (0, c)),
            pl.BlockSpec((BR, 1), lambda r, c: (r, 0)),
        ],
        out_specs=pl.BlockSpec((BR, HH), lambda r, c: (c * GR + r, 0)),
        out_shape=jax.ShapeDtypeStruct((2 * R, HH), _f32),
    )(x_pad, W1, dinv_col)


def _tc_fused_body(a_ref, z_ref, dinv_ref, b_ref, w_ref, o_ref):
    acc = jnp.concatenate([a_ref[0], a_ref[1]], axis=1)
    zc = jnp.concatenate([z_ref[0], z_ref[1]], axis=1)
    dinv = dinv_ref[...]
    act = jnp.maximum(dinv * (acc + zc) + b_ref[...], 0.0)
    o_ref[...] = jnp.dot(act, w_ref[...], preferred_element_type=_f32) * dinv


def _tc_fused(acc, zp, dinv_col, b, W2):
    return pl.pallas_call(
        _tc_fused_body,
        grid=(GR, 2),
        in_specs=[
            pl.BlockSpec((2, BR, HH), lambda r, c: (0, r, 0)),
            pl.BlockSpec((2, BR, HH), lambda r, c: (0, r, 0)),
            pl.BlockSpec((BR, 1), lambda r, c: (r, 0)),
            pl.BlockSpec((1, H), lambda r, c: (0, 0)),
            pl.BlockSpec((H, HH), lambda r, c: (0, c)),
        ],
        out_specs=pl.BlockSpec((BR, HH), lambda r, c: (c * GR + r, 0)),
        out_shape=jax.ShapeDtypeStruct((2 * R, HH), _f32),
    )(acc.reshape(2, R, HH), zp.reshape(2, R, HH), dinv_col,
      b.reshape(1, H), W2)


def _tc_fused3(acc, zp, dinv_col, b, W3p):
    return pl.pallas_call(
        _tc_fused_body,
        grid=(GR, 1),
        in_specs=[
            pl.BlockSpec((2, BR, HH), lambda r, c: (0, r, 0)),
            pl.BlockSpec((2, BR, HH), lambda r, c: (0, r, 0)),
            pl.BlockSpec((BR, 1), lambda r, c: (r, 0)),
            pl.BlockSpec((1, H), lambda r, c: (0, 0)),
            pl.BlockSpec((H, HH), lambda r, c: (0, 0)),
        ],
        out_specs=pl.BlockSpec((BR, HH), lambda r, c: (r, 0)),
        out_shape=jax.ShapeDtypeStruct((R, HH), _f32),
    )(acc.reshape(2, R, HH), zp.reshape(2, R, HH), dinv_col,
      b.reshape(1, H), W3p)


def _tc_final_body(a_ref, z_ref, dinv_ref, b_ref, o_ref):
    sgt = dinv_ref[...] * (a_ref[0] + a_ref[1] + z_ref[...]) + b_ref[...]
    col = lax.broadcasted_iota(_i32, (BR, HH), 1)
    sm = jnp.where(col < C, sgt, -1e30)
    m = jnp.max(sm, axis=1, keepdims=True)
    e = jnp.where(col < C, jnp.exp(sm - m), 0.0)
    lse = jnp.log(jnp.sum(e, axis=1, keepdims=True))
    o_ref[...] = sm - m - lse


def _tc_final(acc3, z3p, dinv_col, b3p):
    return pl.pallas_call(
        _tc_final_body,
        grid=(GR,),
        in_specs=[
            pl.BlockSpec((2, BR, HH), lambda r: (0, r, 0)),
            pl.BlockSpec((BR, HH), lambda r: (r, 0)),
            pl.BlockSpec((BR, 1), lambda r: (r, 0)),
            pl.BlockSpec((1, HH), lambda r: (0, 0)),
        ],
        out_specs=pl.BlockSpec((BR, HH), lambda r: (r, 0)),
        out_shape=jax.ShapeDtypeStruct((R, HH), _f32),
    )(acc3.reshape(2, R, HH), z3p, dinv_col, b3p)


def _slab(v, nch, pad_val):
    per = v.shape[0] // NT
    v = v.reshape(NT, per)
    v = jnp.pad(v, ((0, 0), (0, nch * CH - per)), constant_values=pad_val)
    return v.reshape(NT, nch, CH)


def _slab_idx(v, nch):
    per = v.shape[0] // NT
    npad = nch * CH - per
    padv = N + (jnp.arange(npad, dtype=v.dtype) % (R - N))
    v = v.reshape(NT, per)
    v = jnp.concatenate([v, jnp.broadcast_to(padv, (NT, npad))], axis=1)
    return v.reshape(NT, nch, CH)


def kernel(x, old_edge_index, new_edges, edge_probs, W1, b1, W2, b2, W3, b3):
    osrc = _slab_idx(old_edge_index[0], NCH_OLD)
    odst = _slab_idx(old_edge_index[1], NCH_OLD)
    nsrc = _slab_idx(new_edges[0], NCH_NEW)
    ndst = _slab_idx(new_edges[1], NCH_NEW)
    pslab = _slab(edge_probs, NCH_NEW, 0.0)

    x_pad = jnp.pad(x, ((0, R - N), (0, 0)))
    W3p = jnp.pad(W3, ((0, 0), (0, HH - C)))
    b3p = jnp.pad(b3, (0, HH - C)).reshape(1, HH)

    degp, wslab = _deg_w_kernel(odst, ndst, pslab)
    dinv_col = _tc_dinv(degp).reshape(R, 1)

    z1 = _tc_l1(x_pad, W1, dinv_col)
    a1 = _layer_half_kernel(z1, osrc, odst, nsrc, ndst, wslab)
    z2 = _tc_fused(a1, z1, dinv_col, b1, W2)
    a2 = _layer_half_kernel(z2, osrc, odst, nsrc, ndst, wslab)
    z3 = _tc_fused3(a2, z2, dinv_col, b2, W3p)
    a3 = _l3_kernel(z3, osrc, odst, nsrc, ndst, wslab)
    lp = _tc_final(a3, z3, dinv_col, b3p)

    log_probs = lp[:N, :C]
    updated_edge_index = jnp.concatenate([old_edge_index, new_edges], axis=1)
    return (log_probs, updated_edge_index)
